# baseline (speedup 1.0000x reference)
import functools
import jax
import jax.numpy as jnp
from jax import lax
from jax.experimental import pallas as pl
from jax.experimental.pallas import tpu as pltpu

B, X, C = 1024, 2048, 16
SEG = 4
P, H = 64, 128
XS = X // SEG
BB = 32
NEG = -3.0e38


def _stats_body(u_ref, ds_ref, pde_ref, W1_ref, b1_ref, W2_ref, b2_ref,
                emb_ref, out_ref):
    TL = 128
    NT = X // TL
    lane = jax.lax.broadcasted_iota(jnp.int32, (1, 1, TL), 2)
    last = lane == TL - 1
    p = [u_ref[:, :, k * TL:(k + 1) * TL] for k in range(NT)]
    r = [pltpu.roll(pk, TL - 1, 2) for pk in p]
    dseg = []
    for k in range(NT):
        if k + 1 < NT:
            nb = jnp.where(last, r[k + 1], r[k])
            dseg.append(jnp.abs(nb - p[k]))
        else:
            dseg.append(jnp.abs(r[k] - p[k]) * (1.0 - last.astype(jnp.float32)))
    seg_sum, seg_sq, seg_mn, seg_mx = [], [], [], []
    for s in range(SEG):
        ps = p[4 * s:4 * s + 4]
        a = (ps[0] + ps[1]) + (ps[2] + ps[3])
        q = (ps[0] * ps[0] + ps[1] * ps[1]) + (ps[2] * ps[2] + ps[3] * ps[3])
        seg_mn.append(jnp.minimum(jnp.minimum(ps[0], ps[1]),
                                  jnp.minimum(ps[2], ps[3])))
        seg_mx.append(jnp.maximum(jnp.maximum(ps[0], ps[1]),
                                  jnp.maximum(ps[2], ps[3])))
        seg_sum.append(jnp.sum(a, axis=(1, 2), keepdims=True).reshape(BB, 1))
        seg_sq.append(jnp.sum(q, axis=(1, 2), keepdims=True).reshape(BB, 1))
    tot_sum = (seg_sum[0] + seg_sum[1]) + (seg_sum[2] + seg_sum[3])
    tot_sq = (seg_sq[0] + seg_sq[1]) + (seg_sq[2] + seg_sq[3])
    mn_f = jnp.minimum(jnp.minimum(seg_mn[0], seg_mn[1]),
                       jnp.minimum(seg_mn[2], seg_mn[3]))
    mx_f = jnp.maximum(jnp.maximum(seg_mx[0], seg_mx[1]),
                       jnp.maximum(seg_mx[2], seg_mx[3]))
    mn = jnp.min(mn_f, axis=(1, 2), keepdims=True).reshape(BB, 1)
    mx = jnp.max(mx_f, axis=(1, 2), keepdims=True).reshape(BB, 1)
    ad = [(dseg[4 * k] + dseg[4 * k + 1]) + (dseg[4 * k + 2] + dseg[4 * k + 3])
          for k in range(4)]
    ad_f = (ad[0] + ad[1]) + (ad[2] + ad[3])
    absd = jnp.sum(ad_f, axis=(1, 2), keepdims=True).reshape(BB, 1)

    nf = float(X * C)
    mean = tot_sum * (1.0 / nf)
    var = (tot_sq - tot_sum * tot_sum * (1.0 / nf)) * (1.0 / (nf - 1.0))
    std = jnp.sqrt(jnp.maximum(var, 0.0))
    gmean = absd * (1.0 / float((X - 1) * C))
    sf = float(XS * C)
    smeans = [s_ * (1.0 / sf) for s_ in seg_sum]
    sstds = [jnp.sqrt(jnp.maximum((q - s_ * s_ * (1.0 / sf)) * (1.0 / (sf - 1.0)), 0.0))
             for q, s_ in zip(seg_sq, seg_sum)]

    pde = pde_ref[...]
    ds = ds_ref[...]
    onehot = (ds == jax.lax.broadcasted_iota(jnp.int32, (BB, 4), 1)
              ).astype(jnp.float32)
    dse = jnp.dot(onehot, emb_ref[...], preferred_element_type=jnp.float32)

    feat = jnp.concatenate(
        [mean, std, mn, mx, gmean] + smeans + sstds
        + [pde, dse, jnp.zeros((BB, 3), jnp.float32)], axis=1)
    h = jnp.dot(feat, W1_ref[...], preferred_element_type=jnp.float32) + b1_ref[...]
    h = jnp.maximum(h, 0.0)
    logits = jnp.dot(h, W2_ref[...], preferred_element_type=jnp.float32) + b2_ref[...]
    out_ref[0] = jnp.transpose(logits, (1, 0))


def _make_route():
    from jax.experimental.pallas import tpu_sc as plsc
    info = plsc.get_sparse_core_info()
    NC, NS, L = info.num_cores, info.num_subcores, info.num_lanes
    NW = NC * NS
    RW = B // NW
    G = RW // L
    mesh = plsc.VectorSubcoreMesh(core_axis_name="c", subcore_axis_name="s")

    @functools.partial(
        pl.kernel, mesh=mesh,
        out_type=jax.ShapeDtypeStruct((NW, P, RW), jnp.float32),
        scratch_types=[
            pltpu.VMEM((P, RW), jnp.float32),
            pltpu.VMEM((P, RW), jnp.float32),
        ],
    )
    def route(ltT_hbm, out3_hbm, lbuf, obuf):
        wid = lax.axis_index("s") * NC + lax.axis_index("c")
        pltpu.sync_copy(ltT_hbm.at[wid], lbuf)
        zf = jnp.zeros((L,), jnp.float32)
        for g in range(G):
            sl = pl.ds(g * L, L)
            m1 = jnp.full((L,), NEG, jnp.float32)
            m2 = jnp.full((L,), NEG, jnp.float32)
            i1 = jnp.zeros((L,), jnp.int32)
            i2 = jnp.zeros((L,), jnp.int32)
            l0 = lbuf[0, sl]
            for e in range(P):
                v = lbuf[e, sl]
                gt1 = v > m1
                gt2 = v > m2
                m2 = jnp.where(gt1, m1, jnp.where(gt2, v, m2))
                i2 = jnp.where(gt1, i1, jnp.where(gt2, e, i2))
                m1 = jnp.where(gt1, v, m1)
                i1 = jnp.where(gt1, e, i1)
            has = (i1 == 0) | (i2 == 0)
            i2f = jnp.where(has, i2, 0)
            lt2 = jnp.where(has, m2, l0)
            a = jnp.maximum(m1, lt2)
            e1 = jnp.exp(m1 - a)
            e2 = jnp.exp(lt2 - a)
            inv = 1.0 / (e1 + e2)
            w1 = e1 * inv
            w2 = e2 * inv
            for e in range(P):
                obuf[e, sl] = (jnp.where(i1 == e, w1, zf)
                               + jnp.where(i2f == e, w2, zf))
        pltpu.sync_copy(obuf, out3_hbm.at[wid])

    return route


def kernel(u_state, pde_params, dataset_id, W1, b1, W2, b2, emb):
    ut = jnp.transpose(u_state, (0, 2, 1))
    ds2 = dataset_id.astype(jnp.int32).reshape(B, 1)
    W1p = jnp.concatenate([W1, jnp.zeros((3, H), W1.dtype)], axis=0)
    b1r = b1.reshape(1, H)
    b2r = b2.reshape(1, P)
    NW = B // BB
    ltT = pl.pallas_call(
        _stats_body,
        grid=(NW,),
        in_specs=[
            pl.BlockSpec((BB, C, X), lambda i: (i, 0, 0)),
            pl.BlockSpec((BB, 1), lambda i: (i, 0)),
            pl.BlockSpec((BB, 8), lambda i: (i, 0)),
            pl.BlockSpec((32, H), lambda i: (0, 0)),
            pl.BlockSpec((1, H), lambda i: (0, 0)),
            pl.BlockSpec((H, P), lambda i: (0, 0)),
            pl.BlockSpec((1, P), lambda i: (0, 0)),
            pl.BlockSpec((4, 8), lambda i: (0, 0)),
        ],
        out_specs=pl.BlockSpec((1, P, BB), lambda i: (i, 0, 0)),
        out_shape=jax.ShapeDtypeStruct((NW, P, BB), jnp.float32),
    )(ut, ds2, pde_params, W1p, b1r, W2, b2r, emb)
    out3 = _make_route()(ltT)
    return jnp.transpose(out3, (0, 2, 1)).reshape(B, P)


# speedup vs baseline: 1.0043x; 1.0043x over previous
import functools
import jax
import jax.numpy as jnp
from jax import lax
from jax.experimental import pallas as pl
from jax.experimental.pallas import tpu as pltpu

B, X, C = 1024, 2048, 16
SEG = 4
P, H = 64, 128
XS = X // SEG
BB = 64
NEG = -3.0e38


def _stats_body(u_ref, ds_ref, pde_ref, W1_ref, b1_ref, W2_ref, b2_ref,
                emb_ref, out_ref):
    TL = 128
    NT = X // TL
    lane = jax.lax.broadcasted_iota(jnp.int32, (1, 1, TL), 2)
    last = lane == TL - 1
    p = [u_ref[:, :, k * TL:(k + 1) * TL] for k in range(NT)]
    r = [pltpu.roll(pk, TL - 1, 2) for pk in p]
    dseg = []
    for k in range(NT):
        if k + 1 < NT:
            nb = jnp.where(last, r[k + 1], r[k])
            dseg.append(jnp.abs(nb - p[k]))
        else:
            dseg.append(jnp.abs(r[k] - p[k]) * (1.0 - last.astype(jnp.float32)))
    seg_sum, seg_sq, seg_mn, seg_mx = [], [], [], []
    for s in range(SEG):
        ps = p[4 * s:4 * s + 4]
        a = (ps[0] + ps[1]) + (ps[2] + ps[3])
        q = (ps[0] * ps[0] + ps[1] * ps[1]) + (ps[2] * ps[2] + ps[3] * ps[3])
        seg_mn.append(jnp.minimum(jnp.minimum(ps[0], ps[1]),
                                  jnp.minimum(ps[2], ps[3])))
        seg_mx.append(jnp.maximum(jnp.maximum(ps[0], ps[1]),
                                  jnp.maximum(ps[2], ps[3])))
        seg_sum.append(jnp.sum(a, axis=(1, 2), keepdims=True).reshape(BB, 1))
        seg_sq.append(jnp.sum(q, axis=(1, 2), keepdims=True).reshape(BB, 1))
    tot_sum = (seg_sum[0] + seg_sum[1]) + (seg_sum[2] + seg_sum[3])
    tot_sq = (seg_sq[0] + seg_sq[1]) + (seg_sq[2] + seg_sq[3])
    mn_f = jnp.minimum(jnp.minimum(seg_mn[0], seg_mn[1]),
                       jnp.minimum(seg_mn[2], seg_mn[3]))
    mx_f = jnp.maximum(jnp.maximum(seg_mx[0], seg_mx[1]),
                       jnp.maximum(seg_mx[2], seg_mx[3]))
    mn = jnp.min(mn_f, axis=(1, 2), keepdims=True).reshape(BB, 1)
    mx = jnp.max(mx_f, axis=(1, 2), keepdims=True).reshape(BB, 1)
    ad = [(dseg[4 * k] + dseg[4 * k + 1]) + (dseg[4 * k + 2] + dseg[4 * k + 3])
          for k in range(4)]
    ad_f = (ad[0] + ad[1]) + (ad[2] + ad[3])
    absd = jnp.sum(ad_f, axis=(1, 2), keepdims=True).reshape(BB, 1)

    nf = float(X * C)
    mean = tot_sum * (1.0 / nf)
    var = (tot_sq - tot_sum * tot_sum * (1.0 / nf)) * (1.0 / (nf - 1.0))
    std = jnp.sqrt(jnp.maximum(var, 0.0))
    gmean = absd * (1.0 / float((X - 1) * C))
    sf = float(XS * C)
    smeans = [s_ * (1.0 / sf) for s_ in seg_sum]
    sstds = [jnp.sqrt(jnp.maximum((q - s_ * s_ * (1.0 / sf)) * (1.0 / (sf - 1.0)), 0.0))
             for q, s_ in zip(seg_sq, seg_sum)]

    pde = pde_ref[...]
    ds = ds_ref[...]
    onehot = (ds == jax.lax.broadcasted_iota(jnp.int32, (BB, 4), 1)
              ).astype(jnp.float32)
    dse = jnp.dot(onehot, emb_ref[...], preferred_element_type=jnp.float32)

    feat = jnp.concatenate(
        [mean, std, mn, mx, gmean] + smeans + sstds
        + [pde, dse, jnp.zeros((BB, 3), jnp.float32)], axis=1)
    h = jnp.dot(feat, W1_ref[...], preferred_element_type=jnp.float32) + b1_ref[...]
    h = jnp.maximum(h, 0.0)
    logits = jnp.dot(h, W2_ref[...], preferred_element_type=jnp.float32) + b2_ref[...]
    out_ref[0] = jnp.transpose(logits[:BB // 2], (1, 0))
    out_ref[1] = jnp.transpose(logits[BB // 2:], (1, 0))


def _make_route():
    from jax.experimental.pallas import tpu_sc as plsc
    info = plsc.get_sparse_core_info()
    NC, NS, L = info.num_cores, info.num_subcores, info.num_lanes
    NW = NC * NS
    RW = B // NW
    G = RW // L
    mesh = plsc.VectorSubcoreMesh(core_axis_name="c", subcore_axis_name="s")

    @functools.partial(
        pl.kernel, mesh=mesh,
        out_type=jax.ShapeDtypeStruct((NW, P, RW), jnp.float32),
        scratch_types=[
            pltpu.VMEM((P, RW), jnp.float32),
            pltpu.VMEM((P, RW), jnp.float32),
        ],
    )
    def route(ltT_hbm, out3_hbm, lbuf, obuf):
        wid = lax.axis_index("s") * NC + lax.axis_index("c")
        pltpu.sync_copy(ltT_hbm.at[wid], lbuf)
        zf = jnp.zeros((L,), jnp.float32)
        for g in range(G):
            sl = pl.ds(g * L, L)
            m1 = jnp.full((L,), NEG, jnp.float32)
            m2 = jnp.full((L,), NEG, jnp.float32)
            i1 = jnp.zeros((L,), jnp.int32)
            i2 = jnp.zeros((L,), jnp.int32)
            l0 = lbuf[0, sl]
            for e in range(P):
                v = lbuf[e, sl]
                gt1 = v > m1
                gt2 = v > m2
                m2 = jnp.where(gt1, m1, jnp.where(gt2, v, m2))
                i2 = jnp.where(gt1, i1, jnp.where(gt2, e, i2))
                m1 = jnp.where(gt1, v, m1)
                i1 = jnp.where(gt1, e, i1)
            has = (i1 == 0) | (i2 == 0)
            i2f = jnp.where(has, i2, 0)
            lt2 = jnp.where(has, m2, l0)
            a = jnp.maximum(m1, lt2)
            e1 = jnp.exp(m1 - a)
            e2 = jnp.exp(lt2 - a)
            inv = 1.0 / (e1 + e2)
            w1 = e1 * inv
            w2 = e2 * inv
            for e in range(P):
                obuf[e, sl] = (jnp.where(i1 == e, w1, zf)
                               + jnp.where(i2f == e, w2, zf))
        pltpu.sync_copy(obuf, out3_hbm.at[wid])

    return route


def kernel(u_state, pde_params, dataset_id, W1, b1, W2, b2, emb):
    ut = jnp.transpose(u_state, (0, 2, 1))
    ds2 = dataset_id.astype(jnp.int32).reshape(B, 1)
    W1p = jnp.concatenate([W1, jnp.zeros((3, H), W1.dtype)], axis=0)
    b1r = b1.reshape(1, H)
    b2r = b2.reshape(1, P)
    NW = B // 32
    ltT = pl.pallas_call(
        _stats_body,
        grid=(B // BB,),
        in_specs=[
            pl.BlockSpec((BB, C, X), lambda i: (i, 0, 0)),
            pl.BlockSpec((BB, 1), lambda i: (i, 0)),
            pl.BlockSpec((BB, 8), lambda i: (i, 0)),
            pl.BlockSpec((32, H), lambda i: (0, 0)),
            pl.BlockSpec((1, H), lambda i: (0, 0)),
            pl.BlockSpec((H, P), lambda i: (0, 0)),
            pl.BlockSpec((1, P), lambda i: (0, 0)),
            pl.BlockSpec((4, 8), lambda i: (0, 0)),
        ],
        out_specs=pl.BlockSpec((2, P, 32), lambda i: (i, 0, 0)),
        out_shape=jax.ShapeDtypeStruct((NW, P, 32), jnp.float32),
    )(ut, ds2, pde_params, W1p, b1r, W2, b2r, emb)
    out3 = _make_route()(ltT)
    return jnp.transpose(out3, (0, 2, 1)).reshape(B, P)


# speedup vs baseline: 1.0265x; 1.0221x over previous
import functools
import jax
import jax.numpy as jnp
from jax import lax
from jax.experimental import pallas as pl
from jax.experimental.pallas import tpu as pltpu

B, X, C = 1024, 2048, 16
SEG = 4
P, H = 64, 128
XS = X // SEG
BB = 64
BH = BB // 2
NEG = -3.0e38


def _stats(u_ref):
    TL = 128
    NT = X // TL
    lane = jax.lax.broadcasted_iota(jnp.int32, (1, 1, TL), 2)
    last = lane == TL - 1
    p = [u_ref[:, :, k * TL:(k + 1) * TL] for k in range(NT)]
    r = [pltpu.roll(pk, TL - 1, 2) for pk in p]
    dseg = []
    for k in range(NT):
        if k + 1 < NT:
            nb = jnp.where(last, r[k + 1], r[k])
            dseg.append(jnp.abs(nb - p[k]))
        else:
            dseg.append(jnp.abs(r[k] - p[k]) * (1.0 - last.astype(jnp.float32)))
    seg_sum, seg_sq, seg_mn, seg_mx = [], [], [], []
    for s in range(SEG):
        ps = p[4 * s:4 * s + 4]
        a = (ps[0] + ps[1]) + (ps[2] + ps[3])
        q = (ps[0] * ps[0] + ps[1] * ps[1]) + (ps[2] * ps[2] + ps[3] * ps[3])
        seg_mn.append(jnp.minimum(jnp.minimum(ps[0], ps[1]),
                                  jnp.minimum(ps[2], ps[3])))
        seg_mx.append(jnp.maximum(jnp.maximum(ps[0], ps[1]),
                                  jnp.maximum(ps[2], ps[3])))
        seg_sum.append(jnp.sum(a, axis=(1, 2), keepdims=True).reshape(BH, 1))
        seg_sq.append(jnp.sum(q, axis=(1, 2), keepdims=True).reshape(BH, 1))
    tot_sum = (seg_sum[0] + seg_sum[1]) + (seg_sum[2] + seg_sum[3])
    tot_sq = (seg_sq[0] + seg_sq[1]) + (seg_sq[2] + seg_sq[3])
    mn_f = jnp.minimum(jnp.minimum(seg_mn[0], seg_mn[1]),
                       jnp.minimum(seg_mn[2], seg_mn[3]))
    mx_f = jnp.maximum(jnp.maximum(seg_mx[0], seg_mx[1]),
                       jnp.maximum(seg_mx[2], seg_mx[3]))
    mn = jnp.min(mn_f, axis=(1, 2), keepdims=True).reshape(BH, 1)
    mx = jnp.max(mx_f, axis=(1, 2), keepdims=True).reshape(BH, 1)
    ad = [(dseg[4 * k] + dseg[4 * k + 1]) + (dseg[4 * k + 2] + dseg[4 * k + 3])
          for k in range(4)]
    ad_f = (ad[0] + ad[1]) + (ad[2] + ad[3])
    absd = jnp.sum(ad_f, axis=(1, 2), keepdims=True).reshape(BH, 1)

    nf = float(X * C)
    mean = tot_sum * (1.0 / nf)
    var = (tot_sq - tot_sum * tot_sum * (1.0 / nf)) * (1.0 / (nf - 1.0))
    std = jnp.sqrt(jnp.maximum(var, 0.0))
    gmean = absd * (1.0 / float((X - 1) * C))
    sf = float(XS * C)
    smeans = [s_ * (1.0 / sf) for s_ in seg_sum]
    sstds = [jnp.sqrt(jnp.maximum((q - s_ * s_ * (1.0 / sf)) * (1.0 / (sf - 1.0)), 0.0))
             for q, s_ in zip(seg_sq, seg_sum)]
    return [mean, std, mn, mx, gmean] + smeans + sstds


def _stats_body(ua_ref, ub_ref, ds_ref, pde_ref, W1_ref, b1_ref, W2_ref,
                b2_ref, emb_ref, out_ref):
    sa = _stats(ua_ref)
    sb = _stats(ub_ref)
    stats = [jnp.concatenate([a, b], axis=0) for a, b in zip(sa, sb)]

    pde = pde_ref[...]
    ds = ds_ref[...]
    onehot = (ds == jax.lax.broadcasted_iota(jnp.int32, (BB, 4), 1)
              ).astype(jnp.float32)
    dse = jnp.dot(onehot, emb_ref[...], preferred_element_type=jnp.float32)

    feat = jnp.concatenate(
        stats + [pde, dse, jnp.zeros((BB, 3), jnp.float32)], axis=1)
    h = jnp.dot(feat, W1_ref[...], preferred_element_type=jnp.float32) + b1_ref[...]
    h = jnp.maximum(h, 0.0)
    logits = jnp.dot(h, W2_ref[...], preferred_element_type=jnp.float32) + b2_ref[...]
    out_ref[0] = jnp.transpose(logits[:BH], (1, 0))
    out_ref[1] = jnp.transpose(logits[BH:], (1, 0))


def _make_route():
    from jax.experimental.pallas import tpu_sc as plsc
    info = plsc.get_sparse_core_info()
    NC, NS, L = info.num_cores, info.num_subcores, info.num_lanes
    NW = NC * NS
    RW = B // NW
    G = RW // L
    mesh = plsc.VectorSubcoreMesh(core_axis_name="c", subcore_axis_name="s")

    @functools.partial(
        pl.kernel, mesh=mesh,
        out_type=jax.ShapeDtypeStruct((NW, P, RW), jnp.float32),
        scratch_types=[
            pltpu.VMEM((P, RW), jnp.float32),
            pltpu.VMEM((P, RW), jnp.float32),
        ],
    )
    def route(ltT_hbm, out3_hbm, lbuf, obuf):
        wid = lax.axis_index("s") * NC + lax.axis_index("c")
        pltpu.sync_copy(ltT_hbm.at[wid], lbuf)
        zf = jnp.zeros((L,), jnp.float32)
        for g in range(G):
            sl = pl.ds(g * L, L)
            m1 = jnp.full((L,), NEG, jnp.float32)
            m2 = jnp.full((L,), NEG, jnp.float32)
            i1 = jnp.zeros((L,), jnp.int32)
            i2 = jnp.zeros((L,), jnp.int32)
            l0 = lbuf[0, sl]
            for e in range(P):
                v = lbuf[e, sl]
                gt1 = v > m1
                gt2 = v > m2
                m2 = jnp.where(gt1, m1, jnp.where(gt2, v, m2))
                i2 = jnp.where(gt1, i1, jnp.where(gt2, e, i2))
                m1 = jnp.where(gt1, v, m1)
                i1 = jnp.where(gt1, e, i1)
            has = (i1 == 0) | (i2 == 0)
            i2f = jnp.where(has, i2, 0)
            lt2 = jnp.where(has, m2, l0)
            a = jnp.maximum(m1, lt2)
            e1 = jnp.exp(m1 - a)
            e2 = jnp.exp(lt2 - a)
            inv = 1.0 / (e1 + e2)
            w1 = e1 * inv
            w2 = e2 * inv
            for e in range(P):
                obuf[e, sl] = (jnp.where(i1 == e, w1, zf)
                               + jnp.where(i2f == e, w2, zf))
        pltpu.sync_copy(obuf, out3_hbm.at[wid])

    return route


def kernel(u_state, pde_params, dataset_id, W1, b1, W2, b2, emb):
    ut = jnp.transpose(u_state, (0, 2, 1))
    ds2 = dataset_id.astype(jnp.int32).reshape(B, 1)
    W1p = jnp.concatenate([W1, jnp.zeros((3, H), W1.dtype)], axis=0)
    b1r = b1.reshape(1, H)
    b2r = b2.reshape(1, P)
    NW = B // 32
    ltT = pl.pallas_call(
        _stats_body,
        grid=(B // BB,),
        in_specs=[
            pl.BlockSpec((BH, C, X), lambda i: (2 * i, 0, 0)),
            pl.BlockSpec((BH, C, X), lambda i: (2 * i + 1, 0, 0)),
            pl.BlockSpec((BB, 1), lambda i: (i, 0)),
            pl.BlockSpec((BB, 8), lambda i: (i, 0)),
            pl.BlockSpec((32, H), lambda i: (0, 0)),
            pl.BlockSpec((1, H), lambda i: (0, 0)),
            pl.BlockSpec((H, P), lambda i: (0, 0)),
            pl.BlockSpec((1, P), lambda i: (0, 0)),
            pl.BlockSpec((4, 8), lambda i: (0, 0)),
        ],
        out_specs=pl.BlockSpec((2, P, 32), lambda i: (i, 0, 0)),
        out_shape=jax.ShapeDtypeStruct((NW, P, 32), jnp.float32),
    )(ut, ut, ds2, pde_params, W1p, b1r, W2, b2r, emb)
    out3 = _make_route()(ltT)
    return jnp.transpose(out3, (0, 2, 1)).reshape(B, P)
